# Initial kernel scaffold; baseline (speedup 1.0000x reference)
import jax
import jax.numpy as jnp
from jax.experimental import pallas as pl


def kernel(x1, x2, x3, x4, x5, x6, x7, E1, E2, E3, E4, E5, E6, E7, W1, W2, W3, W4, W5, W6, W7):
    raise NotImplementedError("write your pallas kernel here")


# speedup vs baseline: 1.6001x; 1.6001x over previous
import functools

import jax
import jax.numpy as jnp
from jax import lax
from jax.experimental import pallas as pl
from jax.experimental.pallas import tpu as pltpu
from jax.experimental.pallas import tpu_sc as plsc

_INPUT_DIMS = [185, 102, 108, 136, 51, 136, 1002]
_OUT_DIMS = [14, 11, 11, 12, 8, 12, 32]
_DEEP = 256
_B, _L = 1024, 50
_NF = 7

_VPAD = [(v + 7) // 8 * 8 for v in _INPUT_DIMS]
_OFFS = [0]
for _v in _VPAD[:-1]:
    _OFFS.append(_OFFS[-1] + _v)
_VTOT = _OFFS[-1] + _VPAD[-1]
_KPAD = 32

_NC, _NS = 2, 16
_NW = _NC * _NS
_NROWS = _B * _NF * _L
_B_PER_W = _NROWS // _NW
_CHUNK = 112
_N_CHUNK = _B_PER_W // _CHUNK


def _tables_body(e_ref, w_ref, out_ref):
    for f in range(_NF):
        out_ref[_OFFS[f]:_OFFS[f] + _VPAD[f], :] = jnp.dot(
            e_ref[_OFFS[f]:_OFFS[f] + _VPAD[f], :],
            w_ref[f, :, :],
            preferred_element_type=jnp.float32,
        )


def _make_tables(Ep, Ws):
    return pl.pallas_call(
        _tables_body,
        out_shape=jax.ShapeDtypeStruct((_VTOT, _DEEP), jnp.float32),
    )(Ep, Ws)


_sc_mesh = plsc.VectorSubcoreMesh(core_axis_name="c", subcore_axis_name="s")


@functools.partial(
    pl.kernel,
    mesh=_sc_mesh,
    out_type=jax.ShapeDtypeStruct((_NROWS, _DEEP), jnp.float32),
    scratch_types=[
        pltpu.VMEM((_B_PER_W,), jnp.int32),
        pltpu.VMEM((_CHUNK, _DEEP), jnp.float32),
        pltpu.SemaphoreType.DMA,
    ],
)
def _sc_gather(table_hbm, idx_hbm, out_hbm, idx_v, rows_v, sem):
    wid = lax.axis_index("s") * _NC + lax.axis_index("c")
    base = wid * _B_PER_W
    pltpu.sync_copy(idx_hbm.at[pl.ds(base, _B_PER_W)], idx_v)

    @pl.loop(0, _N_CHUNK)
    def _(c):
        pltpu.async_copy(
            table_hbm.at[idx_v.at[pl.ds(c * _CHUNK, _CHUNK)]],
            rows_v,
            sem,
        ).wait()
        pltpu.sync_copy(rows_v, out_hbm.at[pl.ds(base + c * _CHUNK, _CHUNK)])


def kernel(x1, x2, x3, x4, x5, x6, x7, E1, E2, E3, E4, E5, E6, E7,
           W1, W2, W3, W4, W5, W6, W7):
    xs = [x1, x2, x3, x4, x5, x6, x7]
    Es = [E1, E2, E3, E4, E5, E6, E7]
    Ws = [W1, W2, W3, W4, W5, W6, W7]

    Ep = jnp.concatenate(
        [jnp.pad(E, ((0, vp - v), (0, _KPAD - od)))
         for E, v, vp, od in zip(Es, _INPUT_DIMS, _VPAD, _OUT_DIMS)],
        axis=0,
    )
    Wstk = jnp.stack(
        [jnp.pad(W, ((0, _KPAD - od), (0, 0)))
         for W, od in zip(Ws, _OUT_DIMS)],
        axis=0,
    )
    table = _make_tables(Ep, Wstk)

    idx = jnp.concatenate(
        [x.astype(jnp.int32) + off for x, off in zip(xs, _OFFS)], axis=1
    ).reshape(-1)

    out = _sc_gather(table, idx)
    return out.reshape(_B, _NF * _L, _DEEP)


# speedup vs baseline: 1.6648x; 1.0404x over previous
import functools

import jax
import jax.numpy as jnp
from jax import lax
from jax.experimental import pallas as pl
from jax.experimental.pallas import tpu as pltpu
from jax.experimental.pallas import tpu_sc as plsc

_INPUT_DIMS = [185, 102, 108, 136, 51, 136, 1002]
_OUT_DIMS = [14, 11, 11, 12, 8, 12, 32]
_DEEP = 256
_B, _L = 1024, 50
_NF = 7

_VPAD = [(v + 7) // 8 * 8 for v in _INPUT_DIMS]
_OFFS = [0]
for _v in _VPAD[:-1]:
    _OFFS.append(_OFFS[-1] + _v)
_VTOT = _OFFS[-1] + _VPAD[-1]
_KPAD = 32

_NC, _NS = 2, 16
_NW = _NC * _NS
_NROWS = _B * _NF * _L
_B_PER_W = _NROWS // _NW
_CHUNK = 112
_N_CHUNK = _B_PER_W // _CHUNK


def _tables_body(e_ref, w_ref, out_ref):
    for f in range(_NF):
        out_ref[_OFFS[f]:_OFFS[f] + _VPAD[f], :] = jnp.dot(
            e_ref[_OFFS[f]:_OFFS[f] + _VPAD[f], :],
            w_ref[f, :, :],
            preferred_element_type=jnp.float32,
        )


def _make_tables(Ep, Ws):
    return pl.pallas_call(
        _tables_body,
        out_shape=jax.ShapeDtypeStruct((_VTOT, _DEEP), jnp.float32),
    )(Ep, Ws)


_sc_mesh = plsc.VectorSubcoreMesh(core_axis_name="c", subcore_axis_name="s")


@functools.partial(
    pl.kernel,
    mesh=_sc_mesh,
    out_type=jax.ShapeDtypeStruct((_NROWS, _DEEP), jnp.float32),
    scratch_types=[
        pltpu.VMEM((_B_PER_W,), jnp.int32),
        pltpu.VMEM((_CHUNK, _DEEP), jnp.float32),
        pltpu.VMEM((_CHUNK, _DEEP), jnp.float32),
        pltpu.SemaphoreType.DMA,
        pltpu.SemaphoreType.DMA,
    ],
)
def _sc_gather(table_hbm, idx_hbm, out_hbm, idx_v, rows0, rows1, sem0, sem1):
    wid = lax.axis_index("s") * _NC + lax.axis_index("c")
    base = wid * _B_PER_W
    pltpu.sync_copy(idx_hbm.at[pl.ds(base, _B_PER_W)], idx_v)

    def _start(c, buf, sem):
        return pltpu.async_copy(
            table_hbm.at[idx_v.at[pl.ds(c * _CHUNK, _CHUNK)]], buf, sem
        )

    def _store(c, buf):
        pltpu.sync_copy(buf, out_hbm.at[pl.ds(base + c * _CHUNK, _CHUNK)])

    _start(0, rows0, sem0)

    @pl.loop(0, _N_CHUNK, step=2)
    def _(c):
        _start(c + 1, rows1, sem1)
        pltpu.make_async_copy(
            table_hbm.at[idx_v.at[pl.ds(c * _CHUNK, _CHUNK)]], rows0, sem0
        ).wait()
        _store(c, rows0)

        @pl.when(c + 2 < _N_CHUNK)
        def _():
            _start(c + 2, rows0, sem0)

        pltpu.make_async_copy(
            table_hbm.at[idx_v.at[pl.ds((c + 1) * _CHUNK, _CHUNK)]], rows1, sem1
        ).wait()
        _store(c + 1, rows1)


def kernel(x1, x2, x3, x4, x5, x6, x7, E1, E2, E3, E4, E5, E6, E7,
           W1, W2, W3, W4, W5, W6, W7):
    xs = [x1, x2, x3, x4, x5, x6, x7]
    Es = [E1, E2, E3, E4, E5, E6, E7]
    Ws = [W1, W2, W3, W4, W5, W6, W7]

    Ep = jnp.concatenate(
        [jnp.pad(E, ((0, vp - v), (0, _KPAD - od)))
         for E, v, vp, od in zip(Es, _INPUT_DIMS, _VPAD, _OUT_DIMS)],
        axis=0,
    )
    Wstk = jnp.stack(
        [jnp.pad(W, ((0, _KPAD - od), (0, 0)))
         for W, od in zip(Ws, _OUT_DIMS)],
        axis=0,
    )
    table = _make_tables(Ep, Wstk)

    idx = jnp.concatenate(
        [x.astype(jnp.int32) + off for x, off in zip(xs, _OFFS)], axis=1
    ).reshape(-1)

    out = _sc_gather(table, idx)
    return out.reshape(_B, _NF * _L, _DEEP)


# speedup vs baseline: 1.7949x; 1.0781x over previous
import functools

import jax
import jax.numpy as jnp
from jax import lax
from jax.experimental import pallas as pl
from jax.experimental.pallas import tpu as pltpu
from jax.experimental.pallas import tpu_sc as plsc

_INPUT_DIMS = [185, 102, 108, 136, 51, 136, 1002]
_OUT_DIMS = [14, 11, 11, 12, 8, 12, 32]
_DEEP = 256
_B, _L = 1024, 50
_NF = 7

_VPAD = [(v + 7) // 8 * 8 for v in _INPUT_DIMS]
_OFFS = [0]
for _v in _VPAD[:-1]:
    _OFFS.append(_OFFS[-1] + _v)
_VTOT = _OFFS[-1] + _VPAD[-1]
_KPAD = 32

_NC, _NS = 2, 16
_NW = _NC * _NS
_NROWS = _B * _NF * _L
_B_PER_W = _NROWS // _NW
_CHUNK = 112
_N_CHUNK = _B_PER_W // _CHUNK


def _tables_body(e_ref, w_ref, out_ref):
    for f in range(_NF):
        out_ref[_OFFS[f]:_OFFS[f] + _VPAD[f], :] = jnp.dot(
            e_ref[_OFFS[f]:_OFFS[f] + _VPAD[f], :],
            w_ref[f, :, :],
            preferred_element_type=jnp.float32,
        )


def _make_tables(Ep, Ws):
    return pl.pallas_call(
        _tables_body,
        out_shape=jax.ShapeDtypeStruct((_VTOT, _DEEP), jnp.float32),
    )(Ep, Ws)


_sc_mesh = plsc.VectorSubcoreMesh(core_axis_name="c", subcore_axis_name="s")


@functools.partial(
    pl.kernel,
    mesh=_sc_mesh,
    out_type=jax.ShapeDtypeStruct((_NROWS, _DEEP), jnp.float32),
    scratch_types=[
        pltpu.VMEM((_B_PER_W,), jnp.int32),
        pltpu.VMEM((_CHUNK, _DEEP), jnp.float32),
        pltpu.VMEM((_CHUNK, _DEEP), jnp.float32),
        pltpu.SemaphoreType.DMA,
        pltpu.SemaphoreType.DMA,
    ],
)
def _sc_gather(table_hbm, idx_hbm, out_hbm, idx_v, rows0, rows1, sem0, sem1):
    wid = lax.axis_index("s") * _NC + lax.axis_index("c")
    base = wid * _B_PER_W
    pltpu.sync_copy(idx_hbm.at[pl.ds(base, _B_PER_W)], idx_v)

    def _start(c, buf, sem):
        return pltpu.async_copy(
            table_hbm.at[idx_v.at[pl.ds(c * _CHUNK, _CHUNK)]], buf, sem
        )

    def _store(c, buf):
        pltpu.sync_copy(buf, out_hbm.at[pl.ds(base + c * _CHUNK, _CHUNK)])

    _start(0, rows0, sem0)

    @pl.loop(0, _N_CHUNK, step=2)
    def _(c):
        _start(c + 1, rows1, sem1)
        pltpu.make_async_copy(
            table_hbm.at[idx_v.at[pl.ds(c * _CHUNK, _CHUNK)]], rows0, sem0
        ).wait()
        _store(c, rows0)

        @pl.when(c + 2 < _N_CHUNK)
        def _():
            _start(c + 2, rows0, sem0)

        pltpu.make_async_copy(
            table_hbm.at[idx_v.at[pl.ds((c + 1) * _CHUNK, _CHUNK)]], rows1, sem1
        ).wait()
        _store(c + 1, rows1)


_BPB = 8


def _relayout_body(in_ref, out_ref):
    for j in range(_BPB):
        out_ref[j] = in_ref[pl.ds(j * _NF * _L, _NF * _L), :]


def _relayout(flat):
    return pl.pallas_call(
        _relayout_body,
        grid=(_B // _BPB,),
        in_specs=[pl.BlockSpec((_BPB * _NF * _L, _DEEP), lambda i: (i, 0))],
        out_specs=pl.BlockSpec((_BPB, _NF * _L, _DEEP), lambda i: (i, 0, 0)),
        out_shape=jax.ShapeDtypeStruct((_B, _NF * _L, _DEEP), jnp.float32),
    )(flat)


def kernel(x1, x2, x3, x4, x5, x6, x7, E1, E2, E3, E4, E5, E6, E7,
           W1, W2, W3, W4, W5, W6, W7):
    xs = [x1, x2, x3, x4, x5, x6, x7]
    Es = [E1, E2, E3, E4, E5, E6, E7]
    Ws = [W1, W2, W3, W4, W5, W6, W7]

    Ep = jnp.concatenate(
        [jnp.pad(E, ((0, vp - v), (0, _KPAD - od)))
         for E, v, vp, od in zip(Es, _INPUT_DIMS, _VPAD, _OUT_DIMS)],
        axis=0,
    )
    Wstk = jnp.stack(
        [jnp.pad(W, ((0, _KPAD - od), (0, 0)))
         for W, od in zip(Ws, _OUT_DIMS)],
        axis=0,
    )
    table = _make_tables(Ep, Wstk)

    idx = jnp.concatenate(
        [x.astype(jnp.int32) + off for x, off in zip(xs, _OFFS)], axis=1
    ).reshape(-1)

    out = _sc_gather(table, idx)
    return _relayout(out)
